# Initial kernel scaffold; baseline (speedup 1.0000x reference)
import jax
import jax.numpy as jnp
from jax.experimental import pallas as pl


def kernel(node_features, edge_index, edge_attr, particle_types, params):
    raise NotImplementedError("write your pallas kernel here")


# speedup vs baseline: 2.0161x; 2.0161x over previous
import functools

import jax
import jax.numpy as jnp
from jax import lax
from jax.experimental import pallas as pl
from jax.experimental.pallas import tpu as pltpu
from jax.experimental.pallas import tpu_sc as plsc

N = 10000
E = 320000
H = 128
NODE_F = 30
EDGE_F = 4
OUT = 3
NTYPES = 9
EMB = 16
STEPS = 10

NC = 2
NS = 16
NW = NC * NS
CHUNK = 128
CPW = 80
E_PAD = NW * CPW * CHUNK
N_ACC = 10240

BE = 2048
BN = 2000


@functools.cache
def _sc_kernels():
    mesh = plsc.VectorSubcoreMesh(core_axis_name="c", subcore_axis_name="s")

    @functools.partial(
        pl.kernel,
        mesh=mesh,
        out_type=[jax.ShapeDtypeStruct((E_PAD, H), jnp.float32),
                  jax.ShapeDtypeStruct((E_PAD, H), jnp.float32)],
        scratch_types=[
            pltpu.VMEM((CHUNK,), jnp.int32),
            pltpu.VMEM((CHUNK,), jnp.int32),
            pltpu.VMEM((CHUNK, H), jnp.float32),
            pltpu.VMEM((CHUNK, H), jnp.float32),
            pltpu.SemaphoreType.DMA,
            pltpu.SemaphoreType.DMA,
        ],
    )
    def sc_gather(ps_hbm, pr_hbm, snd_hbm, rcv_hbm, outs_hbm, outr_hbm,
                  idx_s, idx_r, rows_s, rows_r, sem_s, sem_r):
        wid = lax.axis_index("s") * NC + lax.axis_index("c")

        def body(j, carry):
            base = (wid * CPW + j) * CHUNK
            pltpu.sync_copy(snd_hbm.at[pl.ds(base, CHUNK)], idx_s)
            pltpu.sync_copy(rcv_hbm.at[pl.ds(base, CHUNK)], idx_r)
            cs = pltpu.async_copy(ps_hbm.at[idx_s], rows_s, sem_s)
            cr = pltpu.async_copy(pr_hbm.at[idx_r], rows_r, sem_r)
            cs.wait()
            cr.wait()
            pltpu.sync_copy(rows_s, outs_hbm.at[pl.ds(base, CHUNK)])
            pltpu.sync_copy(rows_r, outr_hbm.at[pl.ds(base, CHUNK)])
            return carry

        lax.fori_loop(0, CPW, body, 0)

    @functools.partial(
        pl.kernel,
        mesh=mesh,
        out_type=jax.ShapeDtypeStruct((NC * N_ACC, H), jnp.float32),
        scratch_types=[
            pltpu.VMEM((CPW, CHUNK), jnp.int32),
            pltpu.VMEM((CHUNK, H), jnp.float32),
            pltpu.VMEM_SHARED((N_ACC, H), jnp.float32),
        ],
    )
    def sc_scatter(e_hbm, rcv2d_hbm, zeros_hbm, out_hbm, idx_all, rows, acc):
        cid = lax.axis_index("c")
        sid = lax.axis_index("s")
        wid = cid * NS + sid
        rpt = N_ACC // NS

        pltpu.sync_copy(rcv2d_hbm.at[pl.ds(wid * CPW, CPW)], idx_all)
        pltpu.sync_copy(zeros_hbm.at[pl.ds(sid * rpt, rpt)],
                        acc.at[pl.ds(sid * rpt, rpt)])
        plsc.subcore_barrier()

        def body(j, carry):
            base = (wid * CPW + j) * CHUNK
            pltpu.sync_copy(e_hbm.at[pl.ds(base, CHUNK)], rows)
            pltpu.sync_copy(rows, acc.at[idx_all.at[j]], add=True)
            return carry

        lax.fori_loop(0, CPW, body, 0)
        plsc.subcore_barrier()
        pltpu.sync_copy(acc.at[pl.ds(sid * rpt, rpt)],
                        out_hbm.at[pl.ds(cid * N_ACC + sid * rpt, rpt)])

    return sc_gather, sc_scatter


def _relu(x):
    return jnp.maximum(x, 0.0)


def _ln(o, g, beta):
    mu = jnp.mean(o, axis=-1, keepdims=True)
    var = jnp.mean((o - mu) ** 2, axis=-1, keepdims=True)
    return (o - mu) * lax.rsqrt(var + 1e-5) * g + beta


def _dot(a, b):
    return jnp.dot(a, b, preferred_element_type=jnp.float32)


def _colstats_body(x_ref, out_ref):
    i = pl.program_id(0)

    @pl.when(i == 0)
    def _():
        out_ref[...] = jnp.zeros_like(out_ref)

    x = x_ref[...]
    s = jnp.sum(x, axis=0, keepdims=True)
    s2 = jnp.sum(x * x, axis=0, keepdims=True)
    out_ref[...] += jnp.concatenate([s, s2], axis=0)


def _edge_layer_body(ms_ref, mr_ref, e_ref, w1, b1, w2, b2, w3, b3, g, beta,
                     out_ref):
    e = e_ref[...]
    z = ms_ref[...] + mr_ref[...] + _dot(e, w1[...]) + b1[...]
    h = _relu(z)
    h = _relu(_dot(h, w2[...]) + b2[...])
    o = _dot(h, w3[...]) + b3[...]
    out_ref[...] = e + _ln(o, g[...], beta[...])


def _node_layer_body(x_ref, a0_ref, a1_ref, wx, wa, b1, w2, b2, w3, b3, g,
                     beta, ws, wr, x_out, ps_out, pr_out):
    x = x_ref[...]
    agg = a0_ref[...] + a1_ref[...]
    z = _dot(x, wx[...]) + _dot(agg, wa[...]) + b1[...]
    h = _relu(z)
    h = _relu(_dot(h, w2[...]) + b2[...])
    o = _dot(h, w3[...]) + b3[...]
    xo = x + _ln(o, g[...], beta[...])
    x_out[...] = xo
    ps_out[...] = _dot(xo, ws[...])
    pr_out[...] = _dot(xo, wr[...])


def _node_enc_body(xf_ref, pt_ref, stats_ref, emb_ref, w1x, w1e, b1, w2, b2,
                   w3, b3, g, beta, ws, wr, x_out, ps_out, pr_out):
    s = stats_ref[...]
    mean = s[0:1] * (1.0 / N)
    var = jnp.maximum(s[1:2] * (1.0 / N) - mean * mean, 0.0)
    xin = (xf_ref[...] - mean) * lax.rsqrt(var + 1e-8)
    pt = pt_ref[...]
    onehot = (pt == lax.broadcasted_iota(jnp.int32, (pt.shape[0], NTYPES), 1)
              ).astype(jnp.float32)
    emb = _dot(onehot, emb_ref[...])
    z = _dot(xin, w1x[...]) + _dot(emb, w1e[...]) + b1[...]
    h = _relu(z)
    h = _relu(_dot(h, w2[...]) + b2[...])
    o = _dot(h, w3[...]) + b3[...]
    xo = _ln(o, g[...], beta[...])
    x_out[...] = xo
    ps_out[...] = _dot(xo, ws[...])
    pr_out[...] = _dot(xo, wr[...])


def _edge_enc_body(ea_ref, stats_ref, w1, b1, w2, b2, w3, b3, g, beta,
                   out_ref):
    s = stats_ref[...]
    mean = s[0:1] * (1.0 / E)
    var = jnp.maximum(s[1:2] * (1.0 / E) - mean * mean, 0.0)
    xin = (ea_ref[...] - mean) * lax.rsqrt(var + 1e-8)
    h = _relu(_dot(xin, w1[...]) + b1[...])
    h = _relu(_dot(h, w2[...]) + b2[...])
    o = _dot(h, w3[...]) + b3[...]
    out_ref[...] = _ln(o, g[...], beta[...])


def _dec_body(x_ref, w1, b1, w2, b2, w3, b3, out_ref):
    h = _relu(_dot(x_ref[...], w1[...]) + b1[...])
    h = _relu(_dot(h, w2[...]) + b2[...])
    out_ref[...] = _dot(h, w3[...]) + b3[...]


def _bfull(a):
    return pl.BlockSpec(a.shape, lambda i: (0,) * a.ndim)


def _brows(shape, nb):
    blk = (shape[0] // nb,) + tuple(shape[1:])
    nd = len(shape)

    def imap(i):
        return (i,) + (0,) * (nd - 1)

    return pl.BlockSpec(blk, imap)


def _tc_call(body, row_args, bcast_args, out_shapes, nb):
    in_specs = ([_brows(a.shape, nb) for a in row_args]
                + [_bfull(a) for a in bcast_args])
    out_specs = [_brows(s.shape, nb) for s in out_shapes]
    single = len(out_shapes) == 1
    res = pl.pallas_call(
        body,
        grid=(nb,),
        in_specs=in_specs,
        out_specs=out_specs[0] if single else out_specs,
        out_shape=out_shapes[0] if single else out_shapes,
    )(*row_args, *bcast_args)
    return res


def _colstats(x, nb):
    return pl.pallas_call(
        _colstats_body,
        grid=(nb,),
        in_specs=[_brows(x.shape, nb)],
        out_specs=pl.BlockSpec((2, x.shape[1]), lambda i: (0, 0)),
        out_shape=jax.ShapeDtypeStruct((2, x.shape[1]), jnp.float32),
    )(x)


def _mlp_args(p):
    return (p["W1"], p["b1"].reshape(1, -1), p["W2"], p["b2"].reshape(1, -1),
            p["W3"], p["b3"].reshape(1, -1))


def _ln_args(p):
    return (p["g"].reshape(1, -1), p["beta"].reshape(1, -1))


def kernel(node_features, edge_index, edge_attr, particle_types, params):
    p = params
    snd = edge_index[0]
    rcv = edge_index[1]
    pad = E_PAD - E
    snd_p = jnp.concatenate([snd, jnp.zeros((pad,), jnp.int32)])
    rcv_p = jnp.concatenate([rcv, jnp.zeros((pad,), jnp.int32)])
    rcv_sc = jnp.concatenate(
        [rcv, jnp.full((pad,), N, jnp.int32)]).reshape(NW * CPW, CHUNK)
    ea_p = jnp.concatenate(
        [edge_attr, jnp.zeros((pad, EDGE_F), edge_attr.dtype)], axis=0)
    pt2 = particle_types.reshape(N, 1)
    zeros_acc = jnp.zeros((N_ACC, H), jnp.float32)

    nb_n = N // BN
    nb_e = E_PAD // BE

    nstats = _colstats(node_features, nb_n)
    estats = _colstats(ea_p, nb_e)

    enc_n = p["enc_node"]
    w1n = enc_n["W1"]
    l0w1 = p["layers"][0]["edge"]["W1"]
    xshape = jax.ShapeDtypeStruct((N, H), jnp.float32)
    x, ps, pr = _tc_call(
        _node_enc_body,
        [node_features, pt2],
        [nstats, p["emb"], w1n[:NODE_F], w1n[NODE_F:],
         enc_n["b1"].reshape(1, -1), enc_n["W2"], enc_n["b2"].reshape(1, -1),
         enc_n["W3"], enc_n["b3"].reshape(1, -1), *_ln_args(enc_n),
         l0w1[:H], l0w1[H:2 * H]],
        [xshape, xshape, xshape],
        nb_n,
    )

    e = _tc_call(
        _edge_enc_body,
        [ea_p],
        [estats, *_mlp_args(p["enc_edge"]), *_ln_args(p["enc_edge"])],
        [jax.ShapeDtypeStruct((E_PAD, H), jnp.float32)],
        nb_e,
    )

    for i in range(STEPS):
        lp = p["layers"][i]
        ew = lp["edge"]
        nw_ = lp["node"]
        sc_gather, sc_scatter = _sc_kernels()
        ms, mr = sc_gather(ps, pr, snd_p, rcv_p)
        e = _tc_call(
            _edge_layer_body,
            [ms, mr, e],
            [ew["W1"][2 * H:], ew["b1"].reshape(1, -1), ew["W2"],
             ew["b2"].reshape(1, -1), ew["W3"], ew["b3"].reshape(1, -1),
             *_ln_args(ew)],
            [jax.ShapeDtypeStruct((E_PAD, H), jnp.float32)],
            nb_e,
        )
        aggflat = sc_scatter(e, rcv_sc, zeros_acc)
        a0 = lax.slice(aggflat, (0, 0), (N, H))
        a1 = lax.slice(aggflat, (N_ACC, 0), (N_ACC + N, H))
        nxt = p["layers"][(i + 1) % STEPS]["edge"]["W1"]
        x, ps, pr = _tc_call(
            _node_layer_body,
            [x, a0, a1],
            [nw_["W1"][:H], nw_["W1"][H:], nw_["b1"].reshape(1, -1),
             nw_["W2"], nw_["b2"].reshape(1, -1), nw_["W3"],
             nw_["b3"].reshape(1, -1), *_ln_args(nw_),
             nxt[:H], nxt[H:2 * H]],
            [xshape, xshape, xshape],
            nb_n,
        )

    out = _tc_call(
        _dec_body,
        [x],
        [*_mlp_args(p["dec"])],
        [jax.ShapeDtypeStruct((N, OUT), jnp.float32)],
        nb_n,
    )
    return out


# speedup vs baseline: 2.4499x; 1.2152x over previous
import functools

import jax
import jax.numpy as jnp
from jax import lax
from jax.experimental import pallas as pl
from jax.experimental.pallas import tpu as pltpu
from jax.experimental.pallas import tpu_sc as plsc

N = 10000
E = 320000
H = 128
NODE_F = 30
EDGE_F = 4
OUT = 3
NTYPES = 9
EMB = 16
STEPS = 10

NC = 2
NS = 16
NW = NC * NS
CHUNK = 128
CPW = 80
E_PAD = NW * CPW * CHUNK
N_ACC = 10240

BE = 2048
BN = 2000


@functools.cache
def _sc_kernels():
    mesh = plsc.VectorSubcoreMesh(core_axis_name="c", subcore_axis_name="s")

    @functools.partial(
        pl.kernel,
        mesh=mesh,
        out_type=[jax.ShapeDtypeStruct((E_PAD, H), jnp.float32),
                  jax.ShapeDtypeStruct((E_PAD, H), jnp.float32)],
        scratch_types=[
            pltpu.VMEM((CPW, CHUNK), jnp.int32),
            pltpu.VMEM((CPW, CHUNK), jnp.int32),
            pltpu.VMEM((2, CHUNK, H), jnp.float32),
            pltpu.VMEM((2, CHUNK, H), jnp.float32),
            pltpu.SemaphoreType.DMA((2,)),
            pltpu.SemaphoreType.DMA((2,)),
            pltpu.SemaphoreType.DMA((2,)),
            pltpu.SemaphoreType.DMA((2,)),
        ],
    )
    def sc_gather(ps_hbm, pr_hbm, snd_hbm, rcv_hbm, outs_hbm, outr_hbm,
                  idx_s, idx_r, rows_s, rows_r, gs_sem, gr_sem, ws_sem,
                  wr_sem):
        wid = lax.axis_index("s") * NC + lax.axis_index("c")
        base0 = wid * CPW

        pltpu.sync_copy(snd_hbm.at[pl.ds(base0, CPW)], idx_s)
        pltpu.sync_copy(rcv_hbm.at[pl.ds(base0, CPW)], idx_r)

        def fire(j, b):
            pltpu.async_copy(ps_hbm.at[idx_s.at[j]], rows_s.at[b],
                             gs_sem.at[b])
            pltpu.async_copy(pr_hbm.at[idx_r.at[j]], rows_r.at[b],
                             gr_sem.at[b])

        fire(0, 0)
        fire(1, 1)

        def group(g, carry):
            for b in range(2):
                j = g * 2 + b
                pltpu.make_async_copy(ps_hbm.at[idx_s.at[j]], rows_s.at[b],
                                      gs_sem.at[b]).wait()
                pltpu.make_async_copy(pr_hbm.at[idx_r.at[j]], rows_r.at[b],
                                      gr_sem.at[b]).wait()
                dst = pl.ds((base0 + j) * CHUNK, CHUNK)
                pltpu.async_copy(rows_s.at[b], outs_hbm.at[dst], ws_sem.at[b])
                pltpu.async_copy(rows_r.at[b], outr_hbm.at[dst], wr_sem.at[b])

                @pl.when(j + 2 < CPW)
                def _():
                    pltpu.make_async_copy(rows_s.at[b],
                                          outs_hbm.at[dst],
                                          ws_sem.at[b]).wait()
                    pltpu.make_async_copy(rows_r.at[b],
                                          outr_hbm.at[dst],
                                          wr_sem.at[b]).wait()
                    fire(j + 2, b)
            return carry

        lax.fori_loop(0, CPW // 2, group, 0)
        for b in range(2):
            j = CPW - 2 + b
            dst = pl.ds((base0 + j) * CHUNK, CHUNK)
            pltpu.make_async_copy(rows_s.at[b], outs_hbm.at[dst],
                                  ws_sem.at[b]).wait()
            pltpu.make_async_copy(rows_r.at[b], outr_hbm.at[dst],
                                  wr_sem.at[b]).wait()

    @functools.partial(
        pl.kernel,
        mesh=mesh,
        out_type=jax.ShapeDtypeStruct((NC * N_ACC, H), jnp.float32),
        scratch_types=[
            pltpu.VMEM((CPW, CHUNK), jnp.int32),
            pltpu.VMEM((2, CHUNK, H), jnp.float32),
            pltpu.VMEM_SHARED((N_ACC, H), jnp.float32),
            pltpu.SemaphoreType.DMA((2,)),
        ],
    )
    def sc_scatter(e_hbm, rcv2d_hbm, zeros_hbm, out_hbm, idx_all, rows, acc,
                   lsem):
        cid = lax.axis_index("c")
        sid = lax.axis_index("s")
        wid = cid * NS + sid
        rpt = N_ACC // NS
        base0 = wid * CPW

        pltpu.sync_copy(rcv2d_hbm.at[pl.ds(base0, CPW)], idx_all)
        pltpu.sync_copy(zeros_hbm.at[pl.ds(sid * rpt, rpt)],
                        acc.at[pl.ds(sid * rpt, rpt)])
        plsc.subcore_barrier()

        def load(j, b):
            pltpu.async_copy(e_hbm.at[pl.ds((base0 + j) * CHUNK, CHUNK)],
                             rows.at[b], lsem.at[b])

        load(0, 0)
        load(1, 1)

        def group(g, carry):
            for b in range(2):
                j = g * 2 + b
                pltpu.make_async_copy(
                    e_hbm.at[pl.ds((base0 + j) * CHUNK, CHUNK)],
                    rows.at[b], lsem.at[b]).wait()
                pltpu.sync_copy(rows.at[b], acc.at[idx_all.at[j]], add=True)

                @pl.when(j + 2 < CPW)
                def _():
                    load(j + 2, b)
            return carry

        lax.fori_loop(0, CPW // 2, group, 0)
        plsc.subcore_barrier()
        pltpu.sync_copy(acc.at[pl.ds(sid * rpt, rpt)],
                        out_hbm.at[pl.ds(cid * N_ACC + sid * rpt, rpt)])

    return sc_gather, sc_scatter


def _relu(x):
    return jnp.maximum(x, 0.0)


def _ln(o, g, beta):
    mu = jnp.mean(o, axis=-1, keepdims=True)
    var = jnp.mean((o - mu) ** 2, axis=-1, keepdims=True)
    return (o - mu) * lax.rsqrt(var + 1e-5) * g + beta


def _dot(a, b):
    return jnp.dot(a, b, preferred_element_type=jnp.float32)


def _colstats_body(x_ref, out_ref):
    i = pl.program_id(0)

    @pl.when(i == 0)
    def _():
        out_ref[...] = jnp.zeros_like(out_ref)

    x = x_ref[...]
    s = jnp.sum(x, axis=0, keepdims=True)
    s2 = jnp.sum(x * x, axis=0, keepdims=True)
    out_ref[...] += jnp.concatenate([s, s2], axis=0)


def _edge_layer_body(ms_ref, mr_ref, e_ref, w1, b1, w2, b2, w3, b3, g, beta,
                     out_ref):
    e = e_ref[...]
    z = ms_ref[...] + mr_ref[...] + _dot(e, w1[...]) + b1[...]
    h = _relu(z)
    h = _relu(_dot(h, w2[...]) + b2[...])
    o = _dot(h, w3[...]) + b3[...]
    out_ref[...] = e + _ln(o, g[...], beta[...])


def _node_layer_body(x_ref, a0_ref, a1_ref, wx, wa, b1, w2, b2, w3, b3, g,
                     beta, ws, wr, x_out, ps_out, pr_out):
    x = x_ref[...]
    agg = a0_ref[...] + a1_ref[...]
    z = _dot(x, wx[...]) + _dot(agg, wa[...]) + b1[...]
    h = _relu(z)
    h = _relu(_dot(h, w2[...]) + b2[...])
    o = _dot(h, w3[...]) + b3[...]
    xo = x + _ln(o, g[...], beta[...])
    x_out[...] = xo
    ps_out[...] = _dot(xo, ws[...])
    pr_out[...] = _dot(xo, wr[...])


def _node_enc_body(xf_ref, pt_ref, stats_ref, emb_ref, w1x, w1e, b1, w2, b2,
                   w3, b3, g, beta, ws, wr, x_out, ps_out, pr_out):
    s = stats_ref[...]
    mean = s[0:1] * (1.0 / N)
    var = jnp.maximum(s[1:2] * (1.0 / N) - mean * mean, 0.0)
    xin = (xf_ref[...] - mean) * lax.rsqrt(var + 1e-8)
    pt = pt_ref[...]
    onehot = (pt == lax.broadcasted_iota(jnp.int32, (pt.shape[0], NTYPES), 1)
              ).astype(jnp.float32)
    emb = _dot(onehot, emb_ref[...])
    z = _dot(xin, w1x[...]) + _dot(emb, w1e[...]) + b1[...]
    h = _relu(z)
    h = _relu(_dot(h, w2[...]) + b2[...])
    o = _dot(h, w3[...]) + b3[...]
    xo = _ln(o, g[...], beta[...])
    x_out[...] = xo
    ps_out[...] = _dot(xo, ws[...])
    pr_out[...] = _dot(xo, wr[...])


def _edge_enc_body(ea_ref, stats_ref, w1, b1, w2, b2, w3, b3, g, beta,
                   out_ref):
    s = stats_ref[...]
    mean = s[0:1] * (1.0 / E)
    var = jnp.maximum(s[1:2] * (1.0 / E) - mean * mean, 0.0)
    xin = (ea_ref[...] - mean) * lax.rsqrt(var + 1e-8)
    h = _relu(_dot(xin, w1[...]) + b1[...])
    h = _relu(_dot(h, w2[...]) + b2[...])
    o = _dot(h, w3[...]) + b3[...]
    out_ref[...] = _ln(o, g[...], beta[...])


def _dec_body(x_ref, w1, b1, w2, b2, w3, b3, out_ref):
    h = _relu(_dot(x_ref[...], w1[...]) + b1[...])
    h = _relu(_dot(h, w2[...]) + b2[...])
    out_ref[...] = _dot(h, w3[...]) + b3[...]


def _bfull(a):
    return pl.BlockSpec(a.shape, lambda i: (0,) * a.ndim)


def _brows(shape, nb):
    blk = (shape[0] // nb,) + tuple(shape[1:])
    nd = len(shape)

    def imap(i):
        return (i,) + (0,) * (nd - 1)

    return pl.BlockSpec(blk, imap)


def _tc_call(body, row_args, bcast_args, out_shapes, nb):
    in_specs = ([_brows(a.shape, nb) for a in row_args]
                + [_bfull(a) for a in bcast_args])
    out_specs = [_brows(s.shape, nb) for s in out_shapes]
    single = len(out_shapes) == 1
    res = pl.pallas_call(
        body,
        grid=(nb,),
        in_specs=in_specs,
        out_specs=out_specs[0] if single else out_specs,
        out_shape=out_shapes[0] if single else out_shapes,
    )(*row_args, *bcast_args)
    return res


def _colstats(x, nb):
    return pl.pallas_call(
        _colstats_body,
        grid=(nb,),
        in_specs=[_brows(x.shape, nb)],
        out_specs=pl.BlockSpec((2, x.shape[1]), lambda i: (0, 0)),
        out_shape=jax.ShapeDtypeStruct((2, x.shape[1]), jnp.float32),
    )(x)


def _mlp_args(p):
    return (p["W1"], p["b1"].reshape(1, -1), p["W2"], p["b2"].reshape(1, -1),
            p["W3"], p["b3"].reshape(1, -1))


def _ln_args(p):
    return (p["g"].reshape(1, -1), p["beta"].reshape(1, -1))


def kernel(node_features, edge_index, edge_attr, particle_types, params):
    p = params
    snd = edge_index[0]
    rcv = edge_index[1]
    pad = E_PAD - E
    snd_g = jnp.concatenate(
        [snd, jnp.zeros((pad,), jnp.int32)]).reshape(NW * CPW, CHUNK)
    rcv_g = jnp.concatenate(
        [rcv, jnp.zeros((pad,), jnp.int32)]).reshape(NW * CPW, CHUNK)
    rcv_sc = jnp.concatenate(
        [rcv, jnp.full((pad,), N, jnp.int32)]).reshape(NW * CPW, CHUNK)
    ea_p = jnp.concatenate(
        [edge_attr, jnp.zeros((pad, EDGE_F), edge_attr.dtype)], axis=0)
    pt2 = particle_types.reshape(N, 1)
    zeros_acc = jnp.zeros((N_ACC, H), jnp.float32)

    nb_n = N // BN
    nb_e = E_PAD // BE

    nstats = _colstats(node_features, nb_n)
    estats = _colstats(ea_p, nb_e)

    enc_n = p["enc_node"]
    w1n = enc_n["W1"]
    l0w1 = p["layers"][0]["edge"]["W1"]
    xshape = jax.ShapeDtypeStruct((N, H), jnp.float32)
    x, ps, pr = _tc_call(
        _node_enc_body,
        [node_features, pt2],
        [nstats, p["emb"], w1n[:NODE_F], w1n[NODE_F:],
         enc_n["b1"].reshape(1, -1), enc_n["W2"], enc_n["b2"].reshape(1, -1),
         enc_n["W3"], enc_n["b3"].reshape(1, -1), *_ln_args(enc_n),
         l0w1[:H], l0w1[H:2 * H]],
        [xshape, xshape, xshape],
        nb_n,
    )

    e = _tc_call(
        _edge_enc_body,
        [ea_p],
        [estats, *_mlp_args(p["enc_edge"]), *_ln_args(p["enc_edge"])],
        [jax.ShapeDtypeStruct((E_PAD, H), jnp.float32)],
        nb_e,
    )

    for i in range(STEPS):
        lp = p["layers"][i]
        ew = lp["edge"]
        nw_ = lp["node"]
        sc_gather, sc_scatter = _sc_kernels()
        ms, mr = sc_gather(ps, pr, snd_g, rcv_g)
        e = _tc_call(
            _edge_layer_body,
            [ms, mr, e],
            [ew["W1"][2 * H:], ew["b1"].reshape(1, -1), ew["W2"],
             ew["b2"].reshape(1, -1), ew["W3"], ew["b3"].reshape(1, -1),
             *_ln_args(ew)],
            [jax.ShapeDtypeStruct((E_PAD, H), jnp.float32)],
            nb_e,
        )
        aggflat = sc_scatter(e, rcv_sc, zeros_acc)
        a0 = lax.slice(aggflat, (0, 0), (N, H))
        a1 = lax.slice(aggflat, (N_ACC, 0), (N_ACC + N, H))
        nxt = p["layers"][(i + 1) % STEPS]["edge"]["W1"]
        x, ps, pr = _tc_call(
            _node_layer_body,
            [x, a0, a1],
            [nw_["W1"][:H], nw_["W1"][H:], nw_["b1"].reshape(1, -1),
             nw_["W2"], nw_["b2"].reshape(1, -1), nw_["W3"],
             nw_["b3"].reshape(1, -1), *_ln_args(nw_),
             nxt[:H], nxt[H:2 * H]],
            [xshape, xshape, xshape],
            nb_n,
        )

    out = _tc_call(
        _dec_body,
        [x],
        [*_mlp_args(p["dec"])],
        [jax.ShapeDtypeStruct((N, OUT), jnp.float32)],
        nb_n,
    )
    return out
